# Initial kernel scaffold; baseline (speedup 1.0000x reference)
import functools

import jax
import jax.numpy as jnp
from jax import lax
from jax.experimental import pallas as pl
from jax.experimental.pallas import tpu as pltpu
from jax.experimental.pallas import tpu_sc as plsc

N = 10000
E = 320000
D = 128
DP = 144

NC = 2
NS = 16
NW = NC * NS
EPW = E // NW
CH = 80
NCH = EPW // CH
RPT = N // NS
ZR = 125


def _prep_body(x_ref, a_ref, out_ref):
    xb = x_ref[...]
    av = a_ref[...]
    e = jnp.sum(xb * av, axis=1, keepdims=True)
    w = jnp.exp(e)
    y = xb * w
    pad = jnp.zeros((xb.shape[0], DP - D - 1), jnp.float32)
    out_ref[...] = jnp.concatenate([y, w, pad], axis=1)


def _prep(x, a_row):
    B = 1000
    return pl.pallas_call(
        _prep_body,
        grid=(N // B,),
        in_specs=[
            pl.BlockSpec((B, D), lambda i: (i, 0)),
            pl.BlockSpec((1, D), lambda i: (0, 0)),
        ],
        out_specs=pl.BlockSpec((B, DP), lambda i: (i, 0)),
        out_shape=jax.ShapeDtypeStruct((N, DP), jnp.float32),
    )(x, a_row)


def _edges_body(rows_hbm, cols_hbm, y_hbm, z_hbm, out_hbm,
                rows_v, cols_v, buf_v, zbuf_v, acc_sh, sem):
    c = lax.axis_index("c")
    s = lax.axis_index("s")
    tile = c * NS + s

    pltpu.sync_copy(z_hbm, zbuf_v)
    for k in range(RPT // ZR):
        pltpu.sync_copy(zbuf_v, acc_sh.at[pl.ds(s * RPT + k * ZR, ZR)])
    plsc.subcore_barrier()

    base = tile * NCH
    pltpu.sync_copy(rows_hbm.at[pl.ds(base, NCH)], rows_v)
    pltpu.sync_copy(cols_hbm.at[pl.ds(base, NCH)], cols_v)

    def chunk(j, carry):
        pltpu.async_copy(y_hbm.at[cols_v.at[j]], buf_v, sem).wait()
        pltpu.sync_copy(buf_v, acc_sh.at[rows_v.at[j]], add=True)
        return carry

    lax.fori_loop(0, NCH, chunk, 0)

    plsc.subcore_barrier()
    pltpu.sync_copy(acc_sh.at[pl.ds(s * RPT, RPT)],
                    out_hbm.at[c, pl.ds(s * RPT, RPT)])


_edges = functools.partial(
    pl.kernel,
    out_type=jax.ShapeDtypeStruct((NC, N, DP), jnp.float32),
    mesh=plsc.VectorSubcoreMesh(core_axis_name="c", subcore_axis_name="s"),
    scratch_types=[
        pltpu.VMEM((NCH, CH), jnp.int32),
        pltpu.VMEM((NCH, CH), jnp.int32),
        pltpu.VMEM((CH, DP), jnp.float32),
        pltpu.VMEM((ZR, DP), jnp.float32),
        pltpu.VMEM_SHARED((N, DP), jnp.float32),
        pltpu.SemaphoreType.DMA,
    ],
)(_edges_body)


def _combine_body(p0_ref, p1_ref, out_ref):
    sm = p0_ref[...] + p1_ref[...]
    num = sm[:, :D]
    den = sm[:, D:D + 1]
    out_ref[...] = jnp.where(den > 0, num / den, 0.0)


def _combine(p0, p1):
    B = 1000
    return pl.pallas_call(
        _combine_body,
        grid=(N // B,),
        in_specs=[
            pl.BlockSpec((B, DP), lambda i: (i, 0)),
            pl.BlockSpec((B, DP), lambda i: (i, 0)),
        ],
        out_specs=pl.BlockSpec((B, D), lambda i: (i, 0)),
        out_shape=jax.ShapeDtypeStruct((N, D), jnp.float32),
    )(p0, p1)


def kernel(x, edge_index, a):
    row2d = edge_index[0].astype(jnp.int32).reshape(NW * NCH, CH)
    col2d = edge_index[1].astype(jnp.int32).reshape(NW * NCH, CH)
    a_row = a.reshape(1, D)
    zeros = jnp.zeros((ZR, DP), jnp.float32)

    y_pad = _prep(x, a_row)
    parts = _edges(row2d, col2d, y_pad, zeros)
    return _combine(parts[0], parts[1])


# speedup vs baseline: 12.9478x; 12.9478x over previous
import functools

import jax
import jax.numpy as jnp
from jax import lax
from jax.experimental import pallas as pl
from jax.experimental.pallas import tpu as pltpu
from jax.experimental.pallas import tpu_sc as plsc

N = 10000
E = 320000
D = 128
DP = 144
NP = 10240

NC = 2
NS = 16
NW = NC * NS
CH = 128
NCH = 80
EPW = CH * NCH
EPAD = NW * EPW
RPT = NP // NS


def _prep_body(x_ref, a_ref, out_ref):
    xb = x_ref[...]
    av = a_ref[...]
    e = jnp.sum(xb * av, axis=1, keepdims=True)
    w = jnp.exp(e)
    y = xb * w
    pad = jnp.zeros((xb.shape[0], DP - D - 1), jnp.float32)
    out_ref[...] = jnp.concatenate([y, w, pad], axis=1)


def _prep(x, a_row):
    B = 1000
    return pl.pallas_call(
        _prep_body,
        grid=(N // B,),
        in_specs=[
            pl.BlockSpec((B, D), lambda i: (i, 0)),
            pl.BlockSpec((1, D), lambda i: (0, 0)),
        ],
        out_specs=pl.BlockSpec((B, DP), lambda i: (i, 0)),
        out_shape=jax.ShapeDtypeStruct((N, DP), jnp.float32),
    )(x, a_row)


IB = 8


def _edges_body(rows_hbm, cols_hbm, y_hbm, z_hbm, out_hbm,
                rows_v, cols_v, buf_v, acc_sh, sem):
    c = lax.axis_index("c")
    s = lax.axis_index("s")
    tile = c * NS + s

    pltpu.sync_copy(z_hbm, acc_sh.at[pl.ds(s * RPT, RPT)])
    plsc.subcore_barrier()

    base = tile * NCH

    def outer(jo, carry):
        pltpu.sync_copy(rows_hbm.at[pl.ds(base + jo * IB, IB)], rows_v)
        pltpu.sync_copy(cols_hbm.at[pl.ds(base + jo * IB, IB)], cols_v)
        for ji in range(IB):
            pltpu.async_copy(y_hbm.at[cols_v.at[ji]], buf_v, sem).wait()
            pltpu.sync_copy(buf_v, acc_sh.at[rows_v.at[ji]], add=True)
        return carry

    lax.fori_loop(0, NCH // IB, outer, 0)

    plsc.subcore_barrier()
    pltpu.sync_copy(acc_sh.at[pl.ds(s * RPT, RPT)],
                    out_hbm.at[c, pl.ds(s * RPT, RPT)])


_edges = functools.partial(
    pl.kernel,
    out_type=jax.ShapeDtypeStruct((NC, NP, DP), jnp.float32),
    mesh=plsc.VectorSubcoreMesh(core_axis_name="c", subcore_axis_name="s"),
    compiler_params=pltpu.CompilerParams(use_tc_tiling_on_sc=False,
                                         needs_layout_passes=False),
    scratch_types=[
        pltpu.VMEM((IB, CH), jnp.int32),
        pltpu.VMEM((IB, CH), jnp.int32),
        pltpu.VMEM((CH, DP), jnp.float32),
        pltpu.VMEM_SHARED((NP, DP), jnp.float32),
        pltpu.SemaphoreType.DMA,
    ],
)(_edges_body)


def _combine_body(p0_ref, p1_ref, out_ref):
    sm = p0_ref[...] + p1_ref[...]
    num = sm[:, :D]
    den = sm[:, D:D + 1]
    out_ref[...] = jnp.where(den > 0, num / den, 0.0)


def _combine(p0, p1):
    B = 1024
    return pl.pallas_call(
        _combine_body,
        grid=(NP // B,),
        in_specs=[
            pl.BlockSpec((B, DP), lambda i: (i, 0)),
            pl.BlockSpec((B, DP), lambda i: (i, 0)),
        ],
        out_specs=pl.BlockSpec((B, D), lambda i: (i, 0)),
        out_shape=jax.ShapeDtypeStruct((NP, D), jnp.float32),
    )(p0, p1)


def kernel(x, edge_index, a):
    npad = EPAD - E
    row = jnp.concatenate(
        [edge_index[0].astype(jnp.int32), jnp.full((npad,), N, jnp.int32)])
    col = jnp.concatenate(
        [edge_index[1].astype(jnp.int32), jnp.zeros((npad,), jnp.int32)])
    row2d = row.reshape(NW * NCH, CH)
    col2d = col.reshape(NW * NCH, CH)
    a_row = a.reshape(1, D)
    zeros = jnp.zeros((RPT, DP), jnp.float32)

    y_pad = _prep(x, a_row)
    parts = _edges(row2d, col2d, y_pad, zeros)
    return _combine(parts[0], parts[1])[:N]


# speedup vs baseline: 14.5805x; 1.1261x over previous
import functools

import jax
import jax.numpy as jnp
from jax import lax
from jax.experimental import pallas as pl
from jax.experimental.pallas import tpu as pltpu
from jax.experimental.pallas import tpu_sc as plsc

N = 10000
E = 320000
D = 128
DP = 144
NP = 10240

NC = 2
NS = 16
NW = NC * NS
CH = 64
NCH = 160
IB = 16
NOUT = NCH // (2 * IB)
EPW = CH * NCH
EPAD = NW * EPW
RPT = NP // NS


def _prep_body(x_ref, a_ref, out_ref):
    xb = x_ref[...]
    av = a_ref[...]
    e = jnp.sum(xb * av, axis=1, keepdims=True)
    w = jnp.exp(e)
    y = xb * w
    pad = jnp.zeros((xb.shape[0], DP - D - 1), jnp.float32)
    out_ref[...] = jnp.concatenate([y, w, pad], axis=1)


def _prep(x, a_row):
    B = 1000
    return pl.pallas_call(
        _prep_body,
        grid=(N // B,),
        in_specs=[
            pl.BlockSpec((B, D), lambda i: (i, 0)),
            pl.BlockSpec((1, D), lambda i: (0, 0)),
        ],
        out_specs=pl.BlockSpec((B, DP), lambda i: (i, 0)),
        out_shape=jax.ShapeDtypeStruct((N, DP), jnp.float32),
    )(x, a_row)


def _edges_body(rows_hbm, cols_hbm, y_hbm, z_hbm, out_hbm,
                rows_a, cols_a, rows_b, cols_b, buf0, buf1, acc_sh,
                sem0, sem1):
    c = lax.axis_index("c")
    s = lax.axis_index("s")
    tile = c * NS + s

    pltpu.sync_copy(z_hbm, acc_sh.at[pl.ds(s * RPT, RPT)])
    plsc.subcore_barrier()

    base = tile * NCH
    bufs = (buf0, buf1)
    sems = (sem0, sem1)

    pltpu.sync_copy(rows_hbm.at[pl.ds(base, IB)], rows_a)
    pltpu.sync_copy(cols_hbm.at[pl.ds(base, IB)], cols_a)
    pltpu.async_copy(y_hbm.at[cols_a.at[0]], buf0, sem0)

    def outer(jo, carry):
        b0 = base + jo * 2 * IB
        pltpu.sync_copy(rows_hbm.at[pl.ds(b0 + IB, IB)], rows_b)
        pltpu.sync_copy(cols_hbm.at[pl.ds(b0 + IB, IB)], cols_b)
        for k in range(IB):
            if k < IB - 1:
                pltpu.async_copy(y_hbm.at[cols_a.at[k + 1]],
                                 bufs[(k + 1) % 2], sems[(k + 1) % 2])
            else:
                pltpu.async_copy(y_hbm.at[cols_b.at[0]],
                                 bufs[(k + 1) % 2], sems[(k + 1) % 2])
            pltpu.make_async_copy(y_hbm.at[cols_a.at[k]],
                                  bufs[k % 2], sems[k % 2]).wait()
            pltpu.sync_copy(bufs[k % 2], acc_sh.at[rows_a.at[k]], add=True)

        @pl.when(jo < NOUT - 1)
        def _():
            pltpu.sync_copy(rows_hbm.at[pl.ds(b0 + 2 * IB, IB)], rows_a)
            pltpu.sync_copy(cols_hbm.at[pl.ds(b0 + 2 * IB, IB)], cols_a)

        for k in range(IB, 2 * IB):
            kk = k - IB
            if kk < IB - 1:
                pltpu.async_copy(y_hbm.at[cols_b.at[kk + 1]],
                                 bufs[(k + 1) % 2], sems[(k + 1) % 2])
            else:
                @pl.when(jo < NOUT - 1)
                def _():
                    pltpu.async_copy(y_hbm.at[cols_a.at[0]],
                                     bufs[(k + 1) % 2], sems[(k + 1) % 2])
            pltpu.make_async_copy(y_hbm.at[cols_b.at[kk]],
                                  bufs[k % 2], sems[k % 2]).wait()
            pltpu.sync_copy(bufs[k % 2], acc_sh.at[rows_b.at[kk]], add=True)
        return carry

    lax.fori_loop(0, NOUT, outer, 0)

    plsc.subcore_barrier()
    pltpu.sync_copy(acc_sh.at[pl.ds(s * RPT, RPT)],
                    out_hbm.at[c, pl.ds(s * RPT, RPT)])


_edges = functools.partial(
    pl.kernel,
    out_type=jax.ShapeDtypeStruct((NC, NP, DP), jnp.float32),
    mesh=plsc.VectorSubcoreMesh(core_axis_name="c", subcore_axis_name="s"),
    compiler_params=pltpu.CompilerParams(use_tc_tiling_on_sc=False,
                                         needs_layout_passes=False),
    scratch_types=[
        pltpu.VMEM((IB, CH), jnp.int32),
        pltpu.VMEM((IB, CH), jnp.int32),
        pltpu.VMEM((IB, CH), jnp.int32),
        pltpu.VMEM((IB, CH), jnp.int32),
        pltpu.VMEM((CH, DP), jnp.float32),
        pltpu.VMEM((CH, DP), jnp.float32),
        pltpu.VMEM_SHARED((NP, DP), jnp.float32),
        pltpu.SemaphoreType.DMA,
        pltpu.SemaphoreType.DMA,
    ],
)(_edges_body)


def _combine_body(p0_ref, p1_ref, out_ref):
    sm = p0_ref[...] + p1_ref[...]
    num = sm[:, :D]
    den = sm[:, D:D + 1]
    out_ref[...] = jnp.where(den > 0, num / den, 0.0)


def _combine(p0, p1):
    B = 1024
    return pl.pallas_call(
        _combine_body,
        grid=(NP // B,),
        in_specs=[
            pl.BlockSpec((B, DP), lambda i: (i, 0)),
            pl.BlockSpec((B, DP), lambda i: (i, 0)),
        ],
        out_specs=pl.BlockSpec((B, D), lambda i: (i, 0)),
        out_shape=jax.ShapeDtypeStruct((NP, D), jnp.float32),
    )(p0, p1)


def kernel(x, edge_index, a):
    npad = EPAD - E
    row = jnp.concatenate(
        [edge_index[0].astype(jnp.int32), jnp.full((npad,), N, jnp.int32)])
    col = jnp.concatenate(
        [edge_index[1].astype(jnp.int32), jnp.zeros((npad,), jnp.int32)])
    row2d = row.reshape(NW * NCH, CH)
    col2d = col.reshape(NW * NCH, CH)
    a_row = a.reshape(1, D)
    zeros = jnp.zeros((RPT, DP), jnp.float32)

    y_pad = _prep(x, a_row)
    parts = _edges(row2d, col2d, y_pad, zeros)
    return _combine(parts[0], parts[1])[:N]


# speedup vs baseline: 15.6502x; 1.0734x over previous
import functools

import jax
import jax.numpy as jnp
from jax import lax
from jax.experimental import pallas as pl
from jax.experimental.pallas import tpu as pltpu
from jax.experimental.pallas import tpu_sc as plsc

N = 10000
E = 320000
D = 128
DP = 144
NP = 10240

NC = 2
NS = 16
NW = NC * NS
CH = 64
IB = 8
NCHF = 240
NCHS = 80
NOUTF = NCHF // (2 * IB)
NOUTS = NCHS // (2 * IB)
NF_TOTAL = NS * NCHF
EPAD = NS * (NCHF + NCHS) * CH
RPT = NP // NS


def _prep_body(x_ref, a_ref, out_ref):
    xb = x_ref[...]
    av = a_ref[...]
    e = jnp.sum(xb * av, axis=1, keepdims=True)
    w = jnp.exp(e)
    y = xb * w
    pad = jnp.zeros((xb.shape[0], DP - D - 1), jnp.float32)
    out_ref[...] = jnp.concatenate([y, w, pad], axis=1)


def _prep(x, a_row):
    B = 1000
    return pl.pallas_call(
        _prep_body,
        grid=(N // B,),
        in_specs=[
            pl.BlockSpec((B, D), lambda i: (i, 0)),
            pl.BlockSpec((1, D), lambda i: (0, 0)),
        ],
        out_specs=pl.BlockSpec((B, DP), lambda i: (i, 0)),
        out_shape=jax.ShapeDtypeStruct((N, DP), jnp.float32),
    )(x, a_row)


def _edges_body(rows_hbm, cols_hbm, y_hbm, z_hbm, out_hbm,
                rows_a, cols_a, rows_b, cols_b, buf0, buf1, acc_sh,
                sem0, sem1):
    c = lax.axis_index("c")
    s = lax.axis_index("s")

    pltpu.sync_copy(z_hbm, acc_sh.at[pl.ds(s * RPT, RPT)])
    plsc.subcore_barrier()

    fast = c == 0
    nout = jnp.where(fast, NOUTF, NOUTS)
    base = jnp.where(fast, s * NCHF, NF_TOTAL + s * NCHS)
    bufs = (buf0, buf1)
    sems = (sem0, sem1)

    pltpu.sync_copy(rows_hbm.at[pl.ds(base, IB)], rows_a)
    pltpu.sync_copy(cols_hbm.at[pl.ds(base, IB)], cols_a)
    pltpu.async_copy(y_hbm.at[cols_a.at[0]], buf0, sem0)

    def outer(jo, carry):
        b0 = base + jo * 2 * IB
        pltpu.sync_copy(rows_hbm.at[pl.ds(b0 + IB, IB)], rows_b)
        pltpu.sync_copy(cols_hbm.at[pl.ds(b0 + IB, IB)], cols_b)
        for k in range(IB):
            if k < IB - 1:
                pltpu.async_copy(y_hbm.at[cols_a.at[k + 1]],
                                 bufs[(k + 1) % 2], sems[(k + 1) % 2])
            else:
                pltpu.async_copy(y_hbm.at[cols_b.at[0]],
                                 bufs[(k + 1) % 2], sems[(k + 1) % 2])
            pltpu.make_async_copy(y_hbm.at[cols_a.at[k]],
                                  bufs[k % 2], sems[k % 2]).wait()
            pltpu.sync_copy(bufs[k % 2], acc_sh.at[rows_a.at[k]], add=True)

        @pl.when(jo < nout - 1)
        def _():
            pltpu.sync_copy(rows_hbm.at[pl.ds(b0 + 2 * IB, IB)], rows_a)
            pltpu.sync_copy(cols_hbm.at[pl.ds(b0 + 2 * IB, IB)], cols_a)

        for k in range(IB, 2 * IB):
            kk = k - IB
            if kk < IB - 1:
                pltpu.async_copy(y_hbm.at[cols_b.at[kk + 1]],
                                 bufs[(k + 1) % 2], sems[(k + 1) % 2])
            else:
                @pl.when(jo < nout - 1)
                def _():
                    pltpu.async_copy(y_hbm.at[cols_a.at[0]],
                                     bufs[(k + 1) % 2], sems[(k + 1) % 2])
            pltpu.make_async_copy(y_hbm.at[cols_b.at[kk]],
                                  bufs[k % 2], sems[k % 2]).wait()
            pltpu.sync_copy(bufs[k % 2], acc_sh.at[rows_b.at[kk]], add=True)
        return carry

    lax.fori_loop(0, nout, outer, 0)

    plsc.subcore_barrier()
    pltpu.sync_copy(acc_sh.at[pl.ds(s * RPT, RPT)],
                    out_hbm.at[c, pl.ds(s * RPT, RPT)])


_edges = functools.partial(
    pl.kernel,
    out_type=jax.ShapeDtypeStruct((NC, NP, DP), jnp.float32),
    mesh=plsc.VectorSubcoreMesh(core_axis_name="c", subcore_axis_name="s"),
    compiler_params=pltpu.CompilerParams(use_tc_tiling_on_sc=False,
                                         needs_layout_passes=False),
    scratch_types=[
        pltpu.VMEM((IB, CH), jnp.int32),
        pltpu.VMEM((IB, CH), jnp.int32),
        pltpu.VMEM((IB, CH), jnp.int32),
        pltpu.VMEM((IB, CH), jnp.int32),
        pltpu.VMEM((CH, DP), jnp.float32),
        pltpu.VMEM((CH, DP), jnp.float32),
        pltpu.VMEM_SHARED((NP, DP), jnp.float32),
        pltpu.SemaphoreType.DMA,
        pltpu.SemaphoreType.DMA,
    ],
)(_edges_body)


def _combine_body(p0_ref, p1_ref, out_ref):
    sm = p0_ref[...] + p1_ref[...]
    num = sm[:, :D]
    den = sm[:, D:D + 1]
    out_ref[...] = jnp.where(den > 0, num / den, 0.0)


def _combine(p0, p1):
    B = 1024
    return pl.pallas_call(
        _combine_body,
        grid=(NP // B,),
        in_specs=[
            pl.BlockSpec((B, DP), lambda i: (i, 0)),
            pl.BlockSpec((B, DP), lambda i: (i, 0)),
        ],
        out_specs=pl.BlockSpec((B, D), lambda i: (i, 0)),
        out_shape=jax.ShapeDtypeStruct((NP, D), jnp.float32),
    )(p0, p1)


def kernel(x, edge_index, a):
    npad = EPAD - E
    row = jnp.concatenate(
        [edge_index[0].astype(jnp.int32), jnp.full((npad,), N, jnp.int32)])
    col = jnp.concatenate(
        [edge_index[1].astype(jnp.int32), jnp.zeros((npad,), jnp.int32)])
    row2d = row.reshape(EPAD // CH, CH)
    col2d = col.reshape(EPAD // CH, CH)
    a_row = a.reshape(1, D)
    zeros = jnp.zeros((RPT, DP), jnp.float32)

    y_pad = _prep(x, a_row)
    parts = _edges(row2d, col2d, y_pad, zeros)
    return _combine(parts[0], parts[1])[:N]


# speedup vs baseline: 15.7444x; 1.0060x over previous
import functools

import jax
import jax.numpy as jnp
from jax import lax
from jax.experimental import pallas as pl
from jax.experimental.pallas import tpu as pltpu
from jax.experimental.pallas import tpu_sc as plsc

N = 10000
E = 320000
D = 128
DP = 144
NP = 10240

NC = 2
NS = 16
NW = NC * NS
CH = 64
IB = 8
NCHF = 240
NCHS = 80
NOUTF = NCHF // (2 * IB)
NOUTS = NCHS // (2 * IB)
NF_TOTAL = NS * NCHF
EPAD = NS * (NCHF + NCHS) * CH
RPT = NP // NS


def _prep_body(x_ref, a_ref, out_ref):
    xb = x_ref[...]
    av = a_ref[...]
    e = jnp.sum(xb * av, axis=1, keepdims=True)
    w = jnp.exp(e)
    y = xb * w
    pad = jnp.zeros((xb.shape[0], DP - D - 1), jnp.float32)
    out_ref[...] = jnp.concatenate([y, w, pad], axis=1)


def _prep(x, a_row):
    B = 1000
    return pl.pallas_call(
        _prep_body,
        grid=(N // B,),
        in_specs=[
            pl.BlockSpec((B, D), lambda i: (i, 0)),
            pl.BlockSpec((1, D), lambda i: (0, 0)),
        ],
        out_specs=pl.BlockSpec((B, DP), lambda i: (i, 0)),
        out_shape=jax.ShapeDtypeStruct((N, DP), jnp.float32),
    )(x, a_row)


def _edges_body(rows_hbm, cols_hbm, y_hbm, out_hbm,
                rows_a, cols_a, rows_b, cols_b, buf0, buf1, acc_sh,
                sem0, sem1):
    c = lax.axis_index("c")
    s = lax.axis_index("s")

    z16 = jnp.zeros((16,), jnp.float32)
    for r in range(CH):
        for q in range(DP // 16):
            buf0[r, pl.ds(q * 16, 16)] = z16
    for k in range(RPT // CH):
        pltpu.sync_copy(buf0, acc_sh.at[pl.ds(s * RPT + k * CH, CH)])
    plsc.subcore_barrier()

    fast = c == 0
    nout = jnp.where(fast, NOUTF, NOUTS)
    base = jnp.where(fast, s * NCHF, NF_TOTAL + s * NCHS)
    bufs = (buf0, buf1)
    sems = (sem0, sem1)

    pltpu.sync_copy(rows_hbm.at[pl.ds(base, IB)], rows_a)
    pltpu.sync_copy(cols_hbm.at[pl.ds(base, IB)], cols_a)
    pltpu.async_copy(y_hbm.at[cols_a.at[0]], buf0, sem0)

    def outer(jo, carry):
        b0 = base + jo * 2 * IB
        pltpu.sync_copy(rows_hbm.at[pl.ds(b0 + IB, IB)], rows_b)
        pltpu.sync_copy(cols_hbm.at[pl.ds(b0 + IB, IB)], cols_b)
        for k in range(IB):
            if k < IB - 1:
                pltpu.async_copy(y_hbm.at[cols_a.at[k + 1]],
                                 bufs[(k + 1) % 2], sems[(k + 1) % 2])
            else:
                pltpu.async_copy(y_hbm.at[cols_b.at[0]],
                                 bufs[(k + 1) % 2], sems[(k + 1) % 2])
            pltpu.make_async_copy(y_hbm.at[cols_a.at[k]],
                                  bufs[k % 2], sems[k % 2]).wait()
            pltpu.sync_copy(bufs[k % 2], acc_sh.at[rows_a.at[k]], add=True)

        @pl.when(jo < nout - 1)
        def _():
            pltpu.sync_copy(rows_hbm.at[pl.ds(b0 + 2 * IB, IB)], rows_a)
            pltpu.sync_copy(cols_hbm.at[pl.ds(b0 + 2 * IB, IB)], cols_a)

        for k in range(IB, 2 * IB):
            kk = k - IB
            if kk < IB - 1:
                pltpu.async_copy(y_hbm.at[cols_b.at[kk + 1]],
                                 bufs[(k + 1) % 2], sems[(k + 1) % 2])
            else:
                @pl.when(jo < nout - 1)
                def _():
                    pltpu.async_copy(y_hbm.at[cols_a.at[0]],
                                     bufs[(k + 1) % 2], sems[(k + 1) % 2])
            pltpu.make_async_copy(y_hbm.at[cols_b.at[kk]],
                                  bufs[k % 2], sems[k % 2]).wait()
            pltpu.sync_copy(bufs[k % 2], acc_sh.at[rows_b.at[kk]], add=True)
        return carry

    lax.fori_loop(0, nout, outer, 0)

    plsc.subcore_barrier()
    pltpu.sync_copy(acc_sh.at[pl.ds(s * RPT, RPT)],
                    out_hbm.at[c, pl.ds(s * RPT, RPT)])


_edges = functools.partial(
    pl.kernel,
    out_type=jax.ShapeDtypeStruct((NC, NP, DP), jnp.float32),
    mesh=plsc.VectorSubcoreMesh(core_axis_name="c", subcore_axis_name="s"),
    compiler_params=pltpu.CompilerParams(use_tc_tiling_on_sc=False,
                                         needs_layout_passes=False),
    scratch_types=[
        pltpu.VMEM((IB, CH), jnp.int32),
        pltpu.VMEM((IB, CH), jnp.int32),
        pltpu.VMEM((IB, CH), jnp.int32),
        pltpu.VMEM((IB, CH), jnp.int32),
        pltpu.VMEM((CH, DP), jnp.float32),
        pltpu.VMEM((CH, DP), jnp.float32),
        pltpu.VMEM_SHARED((NP, DP), jnp.float32),
        pltpu.SemaphoreType.DMA,
        pltpu.SemaphoreType.DMA,
    ],
)(_edges_body)


def _combine_body(p0_ref, p1_ref, out_ref):
    sm = p0_ref[...] + p1_ref[...]
    num = sm[:, :D]
    den = sm[:, D:D + 1]
    out_ref[...] = jnp.where(den > 0, num / den, 0.0)


def _combine(p0, p1):
    B = 1024
    return pl.pallas_call(
        _combine_body,
        grid=(NP // B,),
        in_specs=[
            pl.BlockSpec((B, DP), lambda i: (i, 0)),
            pl.BlockSpec((B, DP), lambda i: (i, 0)),
        ],
        out_specs=pl.BlockSpec((B, D), lambda i: (i, 0)),
        out_shape=jax.ShapeDtypeStruct((NP, D), jnp.float32),
    )(p0, p1)


def kernel(x, edge_index, a):
    npad = EPAD - E
    row = jnp.concatenate(
        [edge_index[0].astype(jnp.int32), jnp.full((npad,), N, jnp.int32)])
    col = jnp.concatenate(
        [edge_index[1].astype(jnp.int32), jnp.zeros((npad,), jnp.int32)])
    row2d = row.reshape(EPAD // CH, CH)
    col2d = col.reshape(EPAD // CH, CH)
    a_row = a.reshape(1, D)

    y_pad = _prep(x, a_row)
    parts = _edges(row2d, col2d, y_pad)
    return _combine(parts[0], parts[1])[:N]
